# baseline (speedup 1.0000x reference)
import functools

import jax
import jax.numpy as jnp
from jax import lax
from jax.experimental import pallas as pl
from jax.experimental.pallas import tpu as pltpu
from jax.experimental.pallas import tpu_sc as plsc

DIM = 128
CHUNK = 10240
EBLK = 2000
SG = 128
NSC = 2
NSUB = 16
NB = 8
FIRE = 4
BT = 512


def _sc_mesh():
    return plsc.VectorSubcoreMesh(core_axis_name="c", subcore_axis_name="s")


@functools.lru_cache(maxsize=None)
def _make_seg_sum(E: int, K: int):
    assert E % (NSUB * EBLK) == 0
    EC = E // NSUB
    NBLK = EC // EBLK
    NGT = EBLK // 16
    KPC = -(-K // NSC)
    STRIPE = CHUNK // NSUB

    EPAD = -(-EBLK // SG) * SG
    NSG = EPAD // SG

    @functools.partial(
        pl.kernel,
        out_type=jax.ShapeDtypeStruct((K * CHUNK, DIM), jnp.float32),
        mesh=_sc_mesh(),
        scratch_types=[
            pltpu.VMEM((SG,), jnp.int32),
            pltpu.VMEM((SG,), jnp.int32),
            pltpu.VMEM((EPAD,), jnp.int32),
            pltpu.VMEM((2 * SG, DIM), jnp.float32),
            pltpu.VMEM((16, DIM), jnp.float32),
            pltpu.VMEM_SHARED((CHUNK + 8, DIM), jnp.float32),
            pltpu.SemaphoreType.DMA((2,)),
            pltpu.SemaphoreType.DMA((NB,)),
        ],
    )
    def seg_sum(src_hbm, dst_hbm, x_hbm, out_hbm, gidx0, gidx1, didx,
                stage, zrow, acc, gsem, ssem):
        cid = lax.axis_index("c")
        sid = lax.axis_index("s")
        ebase = sid * EC

        def zfill(r, carry):
            for j in range(DIM // 16):
                zrow[r, pl.ds(j * 16, 16)] = jnp.zeros((16,), jnp.float32)
            return carry

        lax.fori_loop(0, 16, zfill, 0)

        def fire(off, sg, slot):
            gbuf = (gidx0, gidx1)[slot]
            pltpu.sync_copy(src_hbm.at[pl.ds(off + sg * SG, SG)], gbuf)
            pltpu.async_copy(
                x_hbm.at[gbuf], stage.at[pl.ds(slot * SG, SG)], gsem.at[slot])

        def gwait(slot):
            pltpu.make_async_copy(
                x_hbm.at[pl.ds(0, SG)],
                stage.at[pl.ds(slot * SG, SG)], gsem.at[slot]).wait()

        def chunk_body(j, carry):
            k = j * NSC + cid

            @pl.when(k < K)
            def _():
                base = k * CHUNK

                def zacc(t, c2):
                    pltpu.sync_copy(
                        zrow, acc.at[pl.ds(sid * STRIPE + t * 16, 16)])
                    return c2

                lax.fori_loop(0, STRIPE // 16, zacc, 0)
                plsc.subcore_barrier()

                def blk_body(bb, c2):
                    off = ebase + bb * EBLK
                    pltpu.sync_copy(dst_hbm.at[pl.ds(off, EBLK)],
                                    didx.at[pl.ds(0, EBLK)])
                    for t in range((EPAD - EBLK) // 16):
                        didx[pl.ds(EBLK + t * 16, 16)] = (
                            jnp.full((16,), -1, jnp.int32))

                    fire(off, 0, 0)
                    for sg in range(NSG):
                        slot = sg % 2
                        gwait(slot)
                        if sg + 1 < NSG:
                            fire(off, sg + 1, 1 - slot)
                        for t in range(SG // 16):
                            d = didx[pl.ds(sg * SG + t * 16, 16)]
                            dl = d - base
                            m = (dl >= 0) & (dl < CHUNK)
                            dv = jnp.where(m, dl, CHUNK)
                            pltpu.async_copy(
                                stage.at[pl.ds(slot * SG + t * 16, 16)],
                                acc.at[dv], ssem.at[t], add=True)
                        for t in range(SG // 16):
                            pltpu.make_async_copy(
                                stage.at[pl.ds(0, 16)],
                                acc.at[pl.ds(0, 16)], ssem.at[t]).wait()
                    return c2

                lax.fori_loop(0, NBLK, blk_body, 0)

                plsc.subcore_barrier()
                pltpu.sync_copy(
                    acc.at[pl.ds(sid * STRIPE, STRIPE)],
                    out_hbm.at[pl.ds(base + sid * STRIPE, STRIPE)])
                plsc.subcore_barrier()

            return carry

        lax.fori_loop(0, KPC, chunk_body, 0)

    return seg_sum


def _rs(deg):
    return lax.rsqrt(jnp.maximum(deg, 1.0))


def _dot(a, b):
    return jnp.dot(a, b, preferred_element_type=jnp.float32)


def _lmsg_body(x_ref, deg_ref, w0, b0, w1, b1, v0, c0, v1, c1,
               msg_ref, l2l_ref):
    x = x_ref[...]
    h = jnp.maximum(_dot(x, w0[...]) + b0[...], 0.0)
    msg_ref[...] = (_dot(h, w1[...]) + b1[...]) * _rs(deg_ref[...])
    rows = lax.broadcasted_iota(jnp.int32, x.shape, 0)
    even = (rows % 2) == 0
    xs = jnp.where(even, pltpu.roll(x, x.shape[0] - 1, 0), pltpu.roll(x, 1, 0))
    h2 = jnp.maximum(_dot(xs, v0[...]) + c0[...], 0.0)
    l2l_ref[...] = _dot(h2, v1[...]) + c1[...]


def _cmsg_body(x_ref, deg_ref, w0, b0, w1, b1, msg_ref):
    x = x_ref[...]
    h = jnp.maximum(_dot(x, w0[...]) + b0[...], 0.0)
    msg_ref[...] = (_dot(h, w1[...]) + b1[...]) * _rs(deg_ref[...])


def _cupd_body(c_ref, agg_ref, deg_ref, w_ref, b_ref, o_ref):
    a = agg_ref[...] * _rs(deg_ref[...])
    o_ref[...] = (_dot(c_ref[...], w_ref[0:DIM, :])
                  + _dot(a, w_ref[DIM:2 * DIM, :]) + b_ref[...])


def _lupd_body(l_ref, agg_ref, l2l_ref, deg_ref, w_ref, b_ref, o_ref):
    a = agg_ref[...] * _rs(deg_ref[...])
    o_ref[...] = (_dot(l_ref[...], w_ref[0:DIM, :])
                  + _dot(a, w_ref[DIM:2 * DIM, :])
                  + _dot(l2l_ref[...], w_ref[2 * DIM:3 * DIM, :]) + b_ref[...])


def _row_spec():
    return pl.BlockSpec((BT, DIM), lambda i: (i, 0))


def _deg_spec():
    return pl.BlockSpec((BT, 1), lambda i: (i, 0))


def _full_spec(shape):
    return pl.BlockSpec(shape, lambda i: tuple(0 for _ in shape))


def _lmsg(x, deg, w0, b0, w1, b1, v0, c0, v1, c1, interpret=False):
    n = x.shape[0]
    grid = (-(-n // BT),)
    return pl.pallas_call(
        _lmsg_body,
        grid=grid,
        in_specs=[_row_spec(), _deg_spec()] + [
            _full_spec(a.shape) for a in (w0, b0, w1, b1, v0, c0, v1, c1)],
        out_specs=[_row_spec(), _row_spec()],
        out_shape=[jax.ShapeDtypeStruct((n, DIM), jnp.float32)] * 2,
        interpret=interpret,
    )(x, deg, w0, b0, w1, b1, v0, c0, v1, c1)


def _cmsg(x, deg, w0, b0, w1, b1, interpret=False):
    n = x.shape[0]
    grid = (-(-n // BT),)
    return pl.pallas_call(
        _cmsg_body,
        grid=grid,
        in_specs=[_row_spec(), _deg_spec()] + [
            _full_spec(a.shape) for a in (w0, b0, w1, b1)],
        out_specs=_row_spec(),
        out_shape=jax.ShapeDtypeStruct((n, DIM), jnp.float32),
        interpret=interpret,
    )(x, deg, w0, b0, w1, b1)


def _cupd(c, agg_pad, deg, w, b, interpret=False):
    n = c.shape[0]
    grid = (-(-n // BT),)
    return pl.pallas_call(
        _cupd_body,
        grid=grid,
        in_specs=[_row_spec(), _row_spec(), _deg_spec(),
                  _full_spec(w.shape), _full_spec(b.shape)],
        out_specs=_row_spec(),
        out_shape=jax.ShapeDtypeStruct((n, DIM), jnp.float32),
        interpret=interpret,
    )(c, agg_pad, deg, w, b)


def _lupd(x, agg_pad, l2l, deg, w, b, interpret=False):
    n = x.shape[0]
    grid = (-(-n // BT),)
    return pl.pallas_call(
        _lupd_body,
        grid=grid,
        in_specs=[_row_spec(), _row_spec(), _row_spec(), _deg_spec(),
                  _full_spec(w.shape), _full_spec(b.shape)],
        out_specs=_row_spec(),
        out_shape=jax.ShapeDtypeStruct((n, DIM), jnp.float32),
        interpret=interpret,
    )(x, agg_pad, l2l, deg, w, b)


def kernel(l_size, c_size, l_edge_index, c_edge_index, l_emb, c_emb,
           l2c_W, l2c_b, c2l_W, c2l_b, l2l_W, l2l_b,
           cu_W, cu_b, lu_W, lu_b):
    L, D = l_emb.shape
    C = c_emb.shape[0]
    E = l_edge_index.shape[0]
    assert D == DIM
    Kl = -(-L // CHUNK)
    Kc = -(-C // CHUNK)
    assert L % 8 == 0 and C % 8 == 0
    seg_l = _make_seg_sum(E, Kl)
    seg_c = _make_seg_sum(E, Kc)
    Kdl = -(-(L // 8) // CHUNK)
    Kdc = -(-(C // 8) // CHUNK)
    seg_dl = _make_seg_sum(E, Kdl)
    seg_dc = _make_seg_sum(E, Kdc)

    pad = jnp.zeros((SG,), jnp.int32)
    li = jnp.concatenate([l_edge_index.astype(jnp.int32), pad])
    ci = jnp.concatenate([c_edge_index.astype(jnp.int32), pad])
    one_l = (1.0 + (jnp.asarray(l_size) - L)).astype(jnp.float32)
    one_c = (1.0 + (jnp.asarray(c_size) - C)).astype(jnp.float32)

    on8 = jnp.zeros((SG, DIM), jnp.float32).at[
        jnp.arange(8), 16 * jnp.arange(8)].set(1.0)
    degl_raw = seg_dl(jnp.bitwise_and(li, 7), lax.shift_right_logical(li, 3),
                      on8)
    degc_raw = seg_dc(jnp.bitwise_and(ci, 7), lax.shift_right_logical(ci, 3),
                      on8)
    degl = degl_raw.reshape(-1, 16)[:L, :1] * one_l
    degc = degc_raw.reshape(-1, 16)[:C, :1] * one_c

    l_embs = [l_emb]
    c_embs = [c_emb]
    n_iter = l2c_W.shape[0]
    for i in range(n_iter):
        b = lambda a: a.reshape(1, DIM)
        msg_l, l2l_msg = _lmsg(
            l_emb, degl,
            l2c_W[i, 0], b(l2c_b[i, 0]), l2c_W[i, 1], b(l2c_b[i, 1]),
            l2l_W[i, 0], b(l2l_b[i, 0]), l2l_W[i, 1], b(l2l_b[i, 1]))
        msg_c = _cmsg(
            c_emb, degc,
            c2l_W[i, 0], b(c2l_b[i, 0]), c2l_W[i, 1], b(c2l_b[i, 1]))
        aggc_pad = seg_c(li, ci, msg_l)
        aggl_pad = seg_l(ci, li, msg_c)
        c_emb = _cupd(c_emb, aggc_pad, degc, cu_W[i], b(cu_b[i]))
        l_emb = _lupd(l_emb, aggl_pad, l2l_msg, degl, lu_W[i], b(lu_b[i]))
        l_embs.append(l_emb)
        c_embs.append(c_emb)
    return (tuple(l_embs), tuple(c_embs))


# speedup vs baseline: 1.0392x; 1.0392x over previous
import functools

import jax
import jax.numpy as jnp
from jax import lax
from jax.experimental import pallas as pl
from jax.experimental.pallas import tpu as pltpu
from jax.experimental.pallas import tpu_sc as plsc

DIM = 128
CHUNK = 11264
EBLK = 2000
SG = 128
NSC = 2
NSUB = 16
NB = 8
FIRE = 4
BT = 512


def _sc_mesh():
    return plsc.VectorSubcoreMesh(core_axis_name="c", subcore_axis_name="s")


@functools.lru_cache(maxsize=None)
def _make_seg_sum(E: int, K: int):
    assert E % (NSUB * EBLK) == 0
    EC = E // NSUB
    NBLK = EC // EBLK
    NGT = EBLK // 16
    KPC = -(-K // NSC)
    STRIPE = CHUNK // NSUB

    @functools.partial(
        pl.kernel,
        out_type=jax.ShapeDtypeStruct((K * CHUNK, DIM), jnp.float32),
        mesh=_sc_mesh(),
        scratch_types=[
            pltpu.VMEM((EBLK,), jnp.int32),
            pltpu.VMEM((EBLK,), jnp.int32),
            pltpu.VMEM((NB * 16, DIM), jnp.float32),
            pltpu.VMEM((16, DIM), jnp.float32),
            pltpu.VMEM_SHARED((CHUNK + 8, DIM), jnp.float32),
            pltpu.SemaphoreType.DMA((NB,)),
            pltpu.SemaphoreType.DMA((NB,)),
        ],
    )
    def seg_sum(src_hbm, dst_hbm, x_hbm, out_hbm, sidx, didx,
                stage, zrow, acc, gsem, ssem):
        cid = lax.axis_index("c")
        sid = lax.axis_index("s")
        ebase = sid * EC

        def zfill(r, carry):
            for j in range(DIM // 16):
                zrow[r, pl.ds(j * 16, 16)] = jnp.zeros((16,), jnp.float32)
            return carry

        lax.fori_loop(0, 16, zfill, 0)

        def fire(g, slot):
            sv = sidx[pl.ds(g * 16, 16)]
            pltpu.async_copy(
                x_hbm.at[sv], stage.at[pl.ds(slot * 16, 16)], gsem.at[slot])

        def chunk_body(j, carry):
            k = j * NSC + cid

            @pl.when(k < K)
            def _():
                base = k * CHUNK

                def zacc(t, c2):
                    pltpu.sync_copy(
                        zrow, acc.at[pl.ds(sid * STRIPE + t * 16, 16)])
                    return c2

                lax.fori_loop(0, STRIPE // 16, zacc, 0)
                plsc.subcore_barrier()

                def blk_body(bb, c2):
                    off = ebase + bb * EBLK
                    pltpu.sync_copy(src_hbm.at[pl.ds(off, EBLK)], sidx)
                    pltpu.sync_copy(dst_hbm.at[pl.ds(off, EBLK)], didx)

                    for u in range(FIRE):
                        fire(u, u)

                    def gbody(gg, c3):
                        gbase = gg * NB
                        for u in range(NB):
                            g = gbase + u
                            fslot = (u + FIRE) % NB

                            @pl.when(g + FIRE < NGT)
                            def _(g=g, fslot=fslot):
                                @pl.when(g >= NB - FIRE)
                                def _():
                                    pltpu.make_async_copy(
                                        stage.at[pl.ds(fslot * 16, 16)],
                                        acc.at[pl.ds(0, 16)],
                                        ssem.at[fslot]).wait()

                                fire(g + FIRE, fslot)

                            @pl.when(g < NGT)
                            def _(g=g, u=u):
                                pltpu.make_async_copy(
                                    x_hbm.at[pl.ds(0, 16)],
                                    stage.at[pl.ds(u * 16, 16)],
                                    gsem.at[u]).wait()
                                d = didx[pl.ds(g * 16, 16)]
                                dl = d - base
                                m = (dl >= 0) & (dl < CHUNK)
                                dv = jnp.where(m, dl, CHUNK)
                                pltpu.async_copy(
                                    stage.at[pl.ds(u * 16, 16)],
                                    acc.at[dv], ssem.at[u], add=True)

                        return c3

                    lax.fori_loop(0, (NGT + NB - 1) // NB, gbody, 0)

                    for t in range(min(NB, NGT)):
                        pltpu.make_async_copy(
                            stage.at[pl.ds(0, 16)],
                            acc.at[pl.ds(0, 16)],
                            ssem.at[(NGT - 1 - t) % NB]).wait()
                    return c2

                lax.fori_loop(0, NBLK, blk_body, 0)

                plsc.subcore_barrier()
                pltpu.sync_copy(
                    acc.at[pl.ds(sid * STRIPE, STRIPE)],
                    out_hbm.at[pl.ds(base + sid * STRIPE, STRIPE)])
                plsc.subcore_barrier()

            return carry

        lax.fori_loop(0, KPC, chunk_body, 0)

    return seg_sum


def _rs(deg):
    return lax.rsqrt(jnp.maximum(deg, 1.0))


def _dot(a, b):
    return jnp.dot(a, b, preferred_element_type=jnp.float32)


def _lmsg_body(x_ref, deg_ref, w0, b0, w1, b1, v0, c0, v1, c1,
               msg_ref, l2l_ref):
    x = x_ref[...]
    h = jnp.maximum(_dot(x, w0[...]) + b0[...], 0.0)
    msg_ref[...] = (_dot(h, w1[...]) + b1[...]) * _rs(deg_ref[...])
    rows = lax.broadcasted_iota(jnp.int32, x.shape, 0)
    even = (rows % 2) == 0
    xs = jnp.where(even, pltpu.roll(x, x.shape[0] - 1, 0), pltpu.roll(x, 1, 0))
    h2 = jnp.maximum(_dot(xs, v0[...]) + c0[...], 0.0)
    l2l_ref[...] = _dot(h2, v1[...]) + c1[...]


def _cmsg_body(x_ref, deg_ref, w0, b0, w1, b1, msg_ref):
    x = x_ref[...]
    h = jnp.maximum(_dot(x, w0[...]) + b0[...], 0.0)
    msg_ref[...] = (_dot(h, w1[...]) + b1[...]) * _rs(deg_ref[...])


def _cupd_body(c_ref, agg_ref, deg_ref, w_ref, b_ref, o_ref):
    a = agg_ref[...] * _rs(deg_ref[...])
    o_ref[...] = (_dot(c_ref[...], w_ref[0:DIM, :])
                  + _dot(a, w_ref[DIM:2 * DIM, :]) + b_ref[...])


def _lupd_body(l_ref, agg_ref, l2l_ref, deg_ref, w_ref, b_ref, o_ref):
    a = agg_ref[...] * _rs(deg_ref[...])
    o_ref[...] = (_dot(l_ref[...], w_ref[0:DIM, :])
                  + _dot(a, w_ref[DIM:2 * DIM, :])
                  + _dot(l2l_ref[...], w_ref[2 * DIM:3 * DIM, :]) + b_ref[...])


def _row_spec():
    return pl.BlockSpec((BT, DIM), lambda i: (i, 0))


def _deg_spec():
    return pl.BlockSpec((BT, 1), lambda i: (i, 0))


def _full_spec(shape):
    return pl.BlockSpec(shape, lambda i: tuple(0 for _ in shape))


def _lmsg(x, deg, w0, b0, w1, b1, v0, c0, v1, c1, interpret=False):
    n = x.shape[0]
    grid = (-(-n // BT),)
    return pl.pallas_call(
        _lmsg_body,
        grid=grid,
        in_specs=[_row_spec(), _deg_spec()] + [
            _full_spec(a.shape) for a in (w0, b0, w1, b1, v0, c0, v1, c1)],
        out_specs=[_row_spec(), _row_spec()],
        out_shape=[jax.ShapeDtypeStruct((n, DIM), jnp.float32)] * 2,
        interpret=interpret,
    )(x, deg, w0, b0, w1, b1, v0, c0, v1, c1)


def _cmsg(x, deg, w0, b0, w1, b1, interpret=False):
    n = x.shape[0]
    grid = (-(-n // BT),)
    return pl.pallas_call(
        _cmsg_body,
        grid=grid,
        in_specs=[_row_spec(), _deg_spec()] + [
            _full_spec(a.shape) for a in (w0, b0, w1, b1)],
        out_specs=_row_spec(),
        out_shape=jax.ShapeDtypeStruct((n, DIM), jnp.float32),
        interpret=interpret,
    )(x, deg, w0, b0, w1, b1)


def _cupd(c, agg_pad, deg, w, b, interpret=False):
    n = c.shape[0]
    grid = (-(-n // BT),)
    return pl.pallas_call(
        _cupd_body,
        grid=grid,
        in_specs=[_row_spec(), _row_spec(), _deg_spec(),
                  _full_spec(w.shape), _full_spec(b.shape)],
        out_specs=_row_spec(),
        out_shape=jax.ShapeDtypeStruct((n, DIM), jnp.float32),
        interpret=interpret,
    )(c, agg_pad, deg, w, b)


def _lupd(x, agg_pad, l2l, deg, w, b, interpret=False):
    n = x.shape[0]
    grid = (-(-n // BT),)
    return pl.pallas_call(
        _lupd_body,
        grid=grid,
        in_specs=[_row_spec(), _row_spec(), _row_spec(), _deg_spec(),
                  _full_spec(w.shape), _full_spec(b.shape)],
        out_specs=_row_spec(),
        out_shape=jax.ShapeDtypeStruct((n, DIM), jnp.float32),
        interpret=interpret,
    )(x, agg_pad, l2l, deg, w, b)


def kernel(l_size, c_size, l_edge_index, c_edge_index, l_emb, c_emb,
           l2c_W, l2c_b, c2l_W, c2l_b, l2l_W, l2l_b,
           cu_W, cu_b, lu_W, lu_b):
    L, D = l_emb.shape
    C = c_emb.shape[0]
    E = l_edge_index.shape[0]
    assert D == DIM
    Kl = -(-L // CHUNK)
    Kc = -(-C // CHUNK)
    assert L % 8 == 0 and C % 8 == 0
    seg_l = _make_seg_sum(E, Kl)
    seg_c = _make_seg_sum(E, Kc)
    Kdl = -(-(L // 8) // CHUNK)
    Kdc = -(-(C // 8) // CHUNK)
    seg_dl = _make_seg_sum(E, Kdl)
    seg_dc = _make_seg_sum(E, Kdc)

    pad = jnp.zeros((SG,), jnp.int32)
    li = jnp.concatenate([l_edge_index.astype(jnp.int32), pad])
    ci = jnp.concatenate([c_edge_index.astype(jnp.int32), pad])
    one_l = (1.0 + (jnp.asarray(l_size) - L)).astype(jnp.float32)
    one_c = (1.0 + (jnp.asarray(c_size) - C)).astype(jnp.float32)

    on8 = jnp.zeros((SG, DIM), jnp.float32).at[
        jnp.arange(8), 16 * jnp.arange(8)].set(1.0)
    degl_raw = seg_dl(jnp.bitwise_and(li, 7), lax.shift_right_logical(li, 3),
                      on8)
    degc_raw = seg_dc(jnp.bitwise_and(ci, 7), lax.shift_right_logical(ci, 3),
                      on8)
    degl = degl_raw.reshape(-1, 16)[:L, :1] * one_l
    degc = degc_raw.reshape(-1, 16)[:C, :1] * one_c

    l_embs = [l_emb]
    c_embs = [c_emb]
    n_iter = l2c_W.shape[0]
    for i in range(n_iter):
        b = lambda a: a.reshape(1, DIM)
        msg_l, l2l_msg = _lmsg(
            l_emb, degl,
            l2c_W[i, 0], b(l2c_b[i, 0]), l2c_W[i, 1], b(l2c_b[i, 1]),
            l2l_W[i, 0], b(l2l_b[i, 0]), l2l_W[i, 1], b(l2l_b[i, 1]))
        msg_c = _cmsg(
            c_emb, degc,
            c2l_W[i, 0], b(c2l_b[i, 0]), c2l_W[i, 1], b(c2l_b[i, 1]))
        aggc_pad = seg_c(li, ci, msg_l)
        aggl_pad = seg_l(ci, li, msg_c)
        c_emb = _cupd(c_emb, aggc_pad, degc, cu_W[i], b(cu_b[i]))
        l_emb = _lupd(l_emb, aggl_pad, l2l_msg, degl, lu_W[i], b(lu_b[i]))
        l_embs.append(l_emb)
        c_embs.append(c_emb)
    return (tuple(l_embs), tuple(c_embs))
